# Initial kernel scaffold; baseline (speedup 1.0000x reference)
import jax
import jax.numpy as jnp
from jax.experimental import pallas as pl


def kernel(xs, W1, b1, W2, b2, W3, b3, W4, b4, W5, b5, W6, b6):
    raise NotImplementedError("write your pallas kernel here")


# speedup vs baseline: 13.6020x; 13.6020x over previous
import jax
import jax.numpy as jnp
from jax.experimental import pallas as pl

B = 1024
_HI = jax.lax.Precision.HIGHEST
_CN = (((1,), (0,)), ((), ()))


def _body(xs_ref, w1t_ref, b1_ref, w2t_ref, b2_ref, w3t_ref, b3_ref,
          w4t_ref, b4_ref, w5t_ref, b5_ref, w6t_ref, b6_ref,
          xhat_ref, zs_ref):
    x = xs_ref[:]
    sq = jnp.sum(x * x, axis=1, keepdims=True)
    g = jax.lax.dot_general(x, x, (((1,), (1,)), ((), ())), precision=_HI)
    d2 = sq + jnp.transpose(sq) - 2.0 * g
    row = jax.lax.broadcasted_iota(jnp.int32, (B, B), 0)
    col = jax.lax.broadcasted_iota(jnp.int32, (B, B), 1)
    d2 = jnp.where(row == col, jnp.float32(1e9), d2)
    rowmin = jnp.min(d2, axis=1, keepdims=True)
    cand = jnp.where(d2 == rowmin, col, B)
    idx = jnp.min(cand, axis=1, keepdims=True)
    onehot = (col == idx).astype(jnp.float32)
    x0 = jax.lax.dot_general(onehot, x, _CN, precision=_HI)
    v = x - x0

    a1 = jax.lax.dot_general(x0, w1t_ref[:], _CN) + b1_ref[:]
    t1 = jax.lax.dot_general(v, w1t_ref[:], _CN)
    h1 = jnp.maximum(a1, 0.0)
    dt1 = jnp.where(a1 > 0.0, t1, 0.0)

    a2 = jax.lax.dot_general(h1, w2t_ref[:], _CN) + b2_ref[:]
    t2 = jax.lax.dot_general(dt1, w2t_ref[:], _CN)
    h2 = jnp.maximum(a2, 0.0)
    dt2 = jnp.where(a2 > 0.0, t2, 0.0)

    z0 = jax.lax.dot_general(h2, w3t_ref[:], _CN) + b3_ref[:]
    gz = jax.lax.dot_general(dt2, w3t_ref[:], _CN)
    zs = z0 + gz
    zs_ref[:] = zs

    h4 = jnp.maximum(jax.lax.dot_general(zs, w4t_ref[:], _CN) + b4_ref[:], 0.0)
    h5 = jnp.maximum(jax.lax.dot_general(h4, w5t_ref[:], _CN) + b5_ref[:], 0.0)
    xhat_ref[:] = jax.lax.dot_general(h5, w6t_ref[:], _CN) + b6_ref[:]


def kernel(xs, W1, b1, W2, b2, W3, b3, W4, b4, W5, b5, W6, b6):
    d = xs.shape[1]
    call = pl.pallas_call(
        _body,
        out_shape=(
            jax.ShapeDtypeStruct((B, d), jnp.float32),
            jax.ShapeDtypeStruct((B, W3.shape[0]), jnp.float32),
        ),
    )
    x_hats, zs = call(
        xs,
        W1.T, b1.reshape(1, -1), W2.T, b2.reshape(1, -1), W3.T, b3.reshape(1, -1),
        W4.T, b4.reshape(1, -1), W5.T, b5.reshape(1, -1), W6.T, b6.reshape(1, -1),
    )
    return (x_hats, zs)


# speedup vs baseline: 13.7299x; 1.0094x over previous
import jax
import jax.numpy as jnp
from jax.experimental import pallas as pl

B = 1024
_HI = jax.lax.Precision.HIGHEST
_CN = (((1,), (0,)), ((), ()))


def _body(xs_ref, w1t_ref, b1_ref, w2t_ref, b2_ref, w3t_ref, b3_ref,
          w4t_ref, b4_ref, w5t_ref, b5_ref, w6t_ref, b6_ref,
          xhat_ref, zs_ref):
    x = xs_ref[:]
    sq = jnp.sum(x * x, axis=1, keepdims=True)
    g = jax.lax.dot_general(x, x, (((1,), (1,)), ((), ())), precision=_HI)
    d2 = sq + jnp.transpose(sq) - 2.0 * g
    row = jax.lax.broadcasted_iota(jnp.int32, (B, B), 0)
    col = jax.lax.broadcasted_iota(jnp.int32, (B, B), 1)
    d2 = jnp.where(row == col, jnp.float32(1e9), d2)
    rowmin = jnp.min(d2, axis=1, keepdims=True)
    cand = jnp.where(d2 == rowmin, col, B)
    idx = jnp.min(cand, axis=1, keepdims=True)
    onehot = (col == idx).astype(jnp.float32)

    a1_all = jax.lax.dot_general(x, w1t_ref[:], _CN)
    a1_nn = jax.lax.dot_general(onehot, a1_all, _CN, precision=_HI)
    a1 = a1_nn + b1_ref[:]
    t1 = a1_all - a1_nn
    h1 = jnp.maximum(a1, 0.0)
    dt1 = jnp.where(a1 > 0.0, t1, 0.0)

    a2 = jax.lax.dot_general(h1, w2t_ref[:], _CN) + b2_ref[:]
    t2 = jax.lax.dot_general(dt1, w2t_ref[:], _CN)
    h2 = jnp.maximum(a2, 0.0)
    dt2 = jnp.where(a2 > 0.0, t2, 0.0)

    z0 = jax.lax.dot_general(h2, w3t_ref[:], _CN) + b3_ref[:]
    gz = jax.lax.dot_general(dt2, w3t_ref[:], _CN)
    zs = z0 + gz
    zs_ref[:] = zs

    h4 = jnp.maximum(jax.lax.dot_general(zs, w4t_ref[:], _CN) + b4_ref[:], 0.0)
    h5 = jnp.maximum(jax.lax.dot_general(h4, w5t_ref[:], _CN) + b5_ref[:], 0.0)
    xhat_ref[:] = jax.lax.dot_general(h5, w6t_ref[:], _CN) + b6_ref[:]


def kernel(xs, W1, b1, W2, b2, W3, b3, W4, b4, W5, b5, W6, b6):
    d = xs.shape[1]
    call = pl.pallas_call(
        _body,
        out_shape=(
            jax.ShapeDtypeStruct((B, d), jnp.float32),
            jax.ShapeDtypeStruct((B, W3.shape[0]), jnp.float32),
        ),
    )
    x_hats, zs = call(
        xs,
        W1.T, b1.reshape(1, -1), W2.T, b2.reshape(1, -1), W3.T, b3.reshape(1, -1),
        W4.T, b4.reshape(1, -1), W5.T, b5.reshape(1, -1), W6.T, b6.reshape(1, -1),
    )
    return (x_hats, zs)


# speedup vs baseline: 22.0163x; 1.6035x over previous
import jax
import jax.numpy as jnp
from jax.experimental import pallas as pl

B = 1024
_HI = jax.lax.Precision.HIGHEST
_CN = (((1,), (0,)), ((), ()))


def _body(xs_ref, w1t_ref, b1_ref, w2t_ref, b2_ref, w3t_ref, b3_ref,
          w4t_ref, b4_ref, w5t_ref, b5_ref, w6t_ref, b6_ref,
          xhat_ref, zs_ref):
    x = xs_ref[:]
    sq = jnp.sum(x * x, axis=1, keepdims=True)
    x_hi = x.astype(jnp.bfloat16).astype(jnp.float32)
    x_lo = x - x_hi
    cat_a = jnp.concatenate([x_hi, x_lo], axis=1)
    cat_b = jnp.concatenate([x_lo, x_hi], axis=1)
    g = (jax.lax.dot_general(cat_a, cat_a, (((1,), (1,)), ((), ())))
         + jax.lax.dot_general(cat_a, cat_b, (((1,), (1,)), ((), ()))))
    d2 = sq + jnp.transpose(sq) - 2.0 * g
    row = jax.lax.broadcasted_iota(jnp.int32, (B, B), 0)
    col = jax.lax.broadcasted_iota(jnp.int32, (B, B), 1)
    d2 = jnp.where(row == col, jnp.float32(1e9), d2)
    rowmin = jnp.min(d2, axis=1, keepdims=True)
    cand = jnp.where(d2 == rowmin, col, B)
    idx = jnp.min(cand, axis=1, keepdims=True)
    onehot = (col == idx).astype(jnp.float32)

    a1_all = jax.lax.dot_general(x, w1t_ref[:], _CN)
    a_hi = a1_all.astype(jnp.bfloat16).astype(jnp.float32)
    r1 = a1_all - a_hi
    a_mid = r1.astype(jnp.bfloat16).astype(jnp.float32)
    a_lo = r1 - a_mid
    limbs = jnp.concatenate([a_hi, a_mid, a_lo], axis=1)
    g3 = jax.lax.dot_general(onehot, limbs, _CN)
    a1_nn = (g3[:, 0:64] + g3[:, 64:128]) + g3[:, 128:192]
    a1 = a1_nn + b1_ref[:]
    t1 = a1_all - a1_nn
    h1 = jnp.maximum(a1, 0.0)
    dt1 = jnp.where(a1 > 0.0, t1, 0.0)

    a2 = jax.lax.dot_general(h1, w2t_ref[:], _CN) + b2_ref[:]
    t2 = jax.lax.dot_general(dt1, w2t_ref[:], _CN)
    h2 = jnp.maximum(a2, 0.0)
    dt2 = jnp.where(a2 > 0.0, t2, 0.0)

    z0 = jax.lax.dot_general(h2, w3t_ref[:], _CN) + b3_ref[:]
    gz = jax.lax.dot_general(dt2, w3t_ref[:], _CN)
    zs = z0 + gz
    zs_ref[:] = zs

    h4 = jnp.maximum(jax.lax.dot_general(zs, w4t_ref[:], _CN) + b4_ref[:], 0.0)
    h5 = jnp.maximum(jax.lax.dot_general(h4, w5t_ref[:], _CN) + b5_ref[:], 0.0)
    xhat_ref[:] = jax.lax.dot_general(h5, w6t_ref[:], _CN) + b6_ref[:]


def kernel(xs, W1, b1, W2, b2, W3, b3, W4, b4, W5, b5, W6, b6):
    d = xs.shape[1]
    call = pl.pallas_call(
        _body,
        out_shape=(
            jax.ShapeDtypeStruct((B, d), jnp.float32),
            jax.ShapeDtypeStruct((B, W3.shape[0]), jnp.float32),
        ),
    )
    x_hats, zs = call(
        xs,
        W1.T, b1.reshape(1, -1), W2.T, b2.reshape(1, -1), W3.T, b3.reshape(1, -1),
        W4.T, b4.reshape(1, -1), W5.T, b5.reshape(1, -1), W6.T, b6.reshape(1, -1),
    )
    return (x_hats, zs)
